# Initial kernel scaffold; baseline (speedup 1.0000x reference)
import jax
import jax.numpy as jnp
from jax.experimental import pallas as pl


def kernel(x, edge_index, W_src, b_src, W_dst, b_dst):
    raise NotImplementedError("write your pallas kernel here")


# speedup vs baseline: 3.0799x; 3.0799x over previous
import functools

import jax
import jax.numpy as jnp
from jax import lax
from jax.experimental import pallas as pl
from jax.experimental.pallas import tpu as pltpu
from jax.experimental.pallas import tpu_sc as plsc

N_NODES = 10000
D = 128
N_EDGES = 320000
NC = 2
NS = 16
B = 128
NCHUNK = 160
IDXH = 16
E_PAD = NS * NCHUNK * B
NACC = 10240
ZPT = NACC // NS


def _sc_scatter(x, src, dst):
    mesh = plsc.VectorSubcoreMesh(core_axis_name="c", subcore_axis_name="s")

    @functools.partial(
        pl.kernel,
        out_type=jax.ShapeDtypeStruct((NC, NACC, D), jnp.float32),
        mesh=mesh,
        scratch_types=(
            pltpu.VMEM_SHARED((NACC, D), jnp.float32),
            pltpu.VMEM((IDXH, B), jnp.int32),
            pltpu.VMEM((IDXH, B), jnp.int32),
            pltpu.VMEM((B, D), jnp.float32),
            pltpu.SemaphoreType.DMA,
        ),
    )
    def k(x_hbm, src_hbm, dst_hbm, out_hbm,
          s_acc, src_idx, dst_idx, rows, sem):
        cid = lax.axis_index("c")
        sid = lax.axis_index("s")

        zvec = jnp.zeros((16,), jnp.float32)

        def zrow(i, c):
            for g in range(D // 16):
                rows[i, pl.ds(g * 16, 16)] = zvec
            return c

        lax.fori_loop(0, B, zrow, 0)
        base = sid * ZPT
        for kk in range(ZPT // B):
            pltpu.sync_copy(rows, s_acc.at[pl.ds(base + kk * B, B)])

        @pl.when(cid == 1)
        def _():
            ovec = jnp.ones((16,), jnp.float32)

            def orow(i, c):
                for g in range(D // 16):
                    rows[i, pl.ds(g * 16, 16)] = ovec
                return c

            lax.fori_loop(0, B, orow, 0)

        plsc.subcore_barrier()

        def slab0(h, c):
            pltpu.sync_copy(src_hbm.at[sid, pl.ds(h * IDXH, IDXH)], src_idx)
            pltpu.sync_copy(dst_hbm.at[sid, pl.ds(h * IDXH, IDXH)], dst_idx)
            for j in range(IDXH):
                pltpu.async_copy(x_hbm.at[src_idx.at[j]], rows, sem).wait()
                pltpu.sync_copy(rows, s_acc.at[dst_idx.at[j]], add=True)
            return c

        def slab1(h, c):
            pltpu.sync_copy(dst_hbm.at[sid, pl.ds(h * IDXH, IDXH)], dst_idx)
            for j in range(IDXH):
                pltpu.sync_copy(rows, s_acc.at[dst_idx.at[j]], add=True)
            return c

        @pl.when(cid == 0)
        def _():
            lax.fori_loop(0, NCHUNK // IDXH, slab0, 0)

        @pl.when(cid == 1)
        def _():
            lax.fori_loop(0, NCHUNK // IDXH, slab1, 0)

        plsc.subcore_barrier()

        pltpu.sync_copy(s_acc.at[pl.ds(base, ZPT)],
                        out_hbm.at[cid, pl.ds(base, ZPT)])

    return k(x, src, dst)


def _tc_body(x_ref, sc_ref, ws_ref, wd_ref, b_ref, o_ref):
    s = sc_ref[0]
    cnt = sc_ref[1]
    mean = s / jnp.maximum(cnt, 1.0)
    o_ref[...] = (
        jnp.dot(mean, ws_ref[...], preferred_element_type=jnp.float32)
        + jnp.dot(x_ref[...], wd_ref[...], preferred_element_type=jnp.float32)
        + b_ref[...]
    )


def _tc_finish(x, sums_counts, ws_t, wd_t, bias):
    R = 1024
    return pl.pallas_call(
        _tc_body,
        grid=(NACC // R,),
        in_specs=[
            pl.BlockSpec((R, D), lambda i: (i, 0)),
            pl.BlockSpec((NC, R, D), lambda i: (0, i, 0)),
            pl.BlockSpec((D, D), lambda i: (0, 0)),
            pl.BlockSpec((D, D), lambda i: (0, 0)),
            pl.BlockSpec((1, D), lambda i: (0, 0)),
        ],
        out_specs=pl.BlockSpec((R, D), lambda i: (i, 0)),
        out_shape=jax.ShapeDtypeStruct((N_NODES, D), jnp.float32),
    )(x, sums_counts, ws_t, wd_t, bias)


def kernel(x, edge_index, W_src, b_src, W_dst, b_dst):
    npad = E_PAD - N_EDGES
    src = edge_index[0].astype(jnp.int32)
    dst = edge_index[1].astype(jnp.int32)
    src = jnp.concatenate([src, jnp.zeros((npad,), jnp.int32)])
    dst = jnp.concatenate([dst, jnp.full((npad,), N_NODES, jnp.int32)])
    src = src.reshape(NS, NCHUNK, B)
    dst = dst.reshape(NS, NCHUNK, B)
    sums_counts = _sc_scatter(x, src, dst)
    bias = (b_src + b_dst).reshape(1, D)
    return _tc_finish(x, sums_counts, W_src.T, W_dst.T, bias)


# speedup vs baseline: 3.7366x; 1.2132x over previous
import functools

import jax
import jax.numpy as jnp
from jax import lax
from jax.experimental import pallas as pl
from jax.experimental.pallas import tpu as pltpu
from jax.experimental.pallas import tpu_sc as plsc

N_NODES = 10000
D = 128
N_EDGES = 320000
NC = 2
NS = 16
B = 128
NCHUNK = 160
IDXH = 16
E_PAD = NS * NCHUNK * B
NACC = 10240
ZPT = NACC // NS


def _sc_scatter(x, src, dst):
    mesh = plsc.VectorSubcoreMesh(core_axis_name="c", subcore_axis_name="s")

    @functools.partial(
        pl.kernel,
        out_type=jax.ShapeDtypeStruct((NC, NACC, D), jnp.float32),
        mesh=mesh,
        scratch_types=(
            pltpu.VMEM_SHARED((NACC, D), jnp.float32),
            pltpu.VMEM((IDXH, B), jnp.int32),
            pltpu.VMEM((IDXH, B), jnp.int32),
            pltpu.VMEM((2, B, D), jnp.float32),
            pltpu.SemaphoreType.DMA,
            pltpu.SemaphoreType.DMA,
        ),
    )
    def k(x_hbm, src_hbm, dst_hbm, out_hbm,
          s_acc, src_idx, dst_idx, rows, sem0, sem1):
        cid = lax.axis_index("c")
        sid = lax.axis_index("s")

        zvec = jnp.zeros((16,), jnp.float32)

        def zrow(i, c):
            for g in range(D // 16):
                rows[0, i, pl.ds(g * 16, 16)] = zvec
            return c

        lax.fori_loop(0, B, zrow, 0)
        base = sid * ZPT
        for kk in range(ZPT // B):
            pltpu.sync_copy(rows.at[0], s_acc.at[pl.ds(base + kk * B, B)])

        @pl.when(cid == 1)
        def _():
            ovec = jnp.ones((16,), jnp.float32)

            def orow(i, c):
                for g in range(D // 16):
                    rows[0, i, pl.ds(g * 16, 16)] = ovec
                return c

            lax.fori_loop(0, B, orow, 0)

        plsc.subcore_barrier()

        sems = (sem0, sem1)

        def slab0(h, c):
            pltpu.sync_copy(src_hbm.at[sid, pl.ds(h * IDXH, IDXH)], src_idx)
            pltpu.sync_copy(dst_hbm.at[sid, pl.ds(h * IDXH, IDXH)], dst_idx)
            cps = [None, None]
            cps[0] = pltpu.async_copy(
                x_hbm.at[src_idx.at[0]], rows.at[0], sem0)
            for j in range(IDXH):
                b = j % 2
                if j + 1 < IDXH:
                    nb = (j + 1) % 2
                    cps[nb] = pltpu.async_copy(
                        x_hbm.at[src_idx.at[j + 1]], rows.at[nb], sems[nb])
                cps[b].wait()
                pltpu.sync_copy(rows.at[b], s_acc.at[dst_idx.at[j]], add=True)
            return c

        def slab1(h, c):
            pltpu.sync_copy(dst_hbm.at[sid, pl.ds(h * IDXH, IDXH)], dst_idx)
            for j in range(IDXH):
                pltpu.sync_copy(rows.at[0], s_acc.at[dst_idx.at[j]], add=True)
            return c

        @pl.when(cid == 0)
        def _():
            lax.fori_loop(0, NCHUNK // IDXH, slab0, 0)

        @pl.when(cid == 1)
        def _():
            lax.fori_loop(0, NCHUNK // IDXH, slab1, 0)

        plsc.subcore_barrier()

        pltpu.sync_copy(s_acc.at[pl.ds(base, ZPT)],
                        out_hbm.at[cid, pl.ds(base, ZPT)])

    return k(x, src, dst)


def _tc_body(x_ref, sc_ref, ws_ref, wd_ref, b_ref, o_ref):
    s = sc_ref[0]
    cnt = sc_ref[1]
    mean = s / jnp.maximum(cnt, 1.0)
    o_ref[...] = (
        jnp.dot(mean, ws_ref[...], preferred_element_type=jnp.float32)
        + jnp.dot(x_ref[...], wd_ref[...], preferred_element_type=jnp.float32)
        + b_ref[...]
    )


def _tc_finish(x, sums_counts, ws_t, wd_t, bias):
    R = 1024
    return pl.pallas_call(
        _tc_body,
        grid=(NACC // R,),
        in_specs=[
            pl.BlockSpec((R, D), lambda i: (i, 0)),
            pl.BlockSpec((NC, R, D), lambda i: (0, i, 0)),
            pl.BlockSpec((D, D), lambda i: (0, 0)),
            pl.BlockSpec((D, D), lambda i: (0, 0)),
            pl.BlockSpec((1, D), lambda i: (0, 0)),
        ],
        out_specs=pl.BlockSpec((R, D), lambda i: (i, 0)),
        out_shape=jax.ShapeDtypeStruct((N_NODES, D), jnp.float32),
    )(x, sums_counts, ws_t, wd_t, bias)


def kernel(x, edge_index, W_src, b_src, W_dst, b_dst):
    npad = E_PAD - N_EDGES
    src = edge_index[0].astype(jnp.int32)
    dst = edge_index[1].astype(jnp.int32)
    src = jnp.concatenate([src, jnp.zeros((npad,), jnp.int32)])
    dst = jnp.concatenate([dst, jnp.full((npad,), N_NODES, jnp.int32)])
    src = src.reshape(NS, NCHUNK, B)
    dst = dst.reshape(NS, NCHUNK, B)
    sums_counts = _sc_scatter(x, src, dst)
    bias = (b_src + b_dst).reshape(1, D)
    return _tc_finish(x, sums_counts, W_src.T, W_dst.T, bias)


# speedup vs baseline: 3.8917x; 1.0415x over previous
import functools

import jax
import jax.numpy as jnp
from jax import lax
from jax.experimental import pallas as pl
from jax.experimental.pallas import tpu as pltpu
from jax.experimental.pallas import tpu_sc as plsc

N_NODES = 10000
D = 128
N_EDGES = 320000
NC = 2
NS = 16
B = 128
NCHUNK = 160
IDXH = 16
NSLAB = NCHUNK // IDXH
SPLIT = 6
E_PAD = NS * NCHUNK * B
NACC = 10240
ZPT = NACC // NS


def _sc_scatter(x, src, dst):
    mesh = plsc.VectorSubcoreMesh(core_axis_name="c", subcore_axis_name="s")

    @functools.partial(
        pl.kernel,
        out_type=jax.ShapeDtypeStruct((3, NACC, D), jnp.float32),
        mesh=mesh,
        scratch_types=(
            pltpu.VMEM_SHARED((NACC, D), jnp.float32),
            pltpu.VMEM((IDXH, B), jnp.int32),
            pltpu.VMEM((IDXH, B), jnp.int32),
            pltpu.VMEM((2, B, D), jnp.float32),
            pltpu.SemaphoreType.DMA,
            pltpu.SemaphoreType.DMA,
        ),
    )
    def k(x_hbm, src_hbm, dst_hbm, out_hbm,
          s_acc, src_idx, dst_idx, rows, sem0, sem1):
        cid = lax.axis_index("c")
        sid = lax.axis_index("s")

        zvec = jnp.zeros((16,), jnp.float32)

        def zrow(i, c):
            for g in range(D // 16):
                rows[0, i, pl.ds(g * 16, 16)] = zvec
            return c

        lax.fori_loop(0, B, zrow, 0)
        base = sid * ZPT
        for kk in range(ZPT // B):
            pltpu.sync_copy(rows.at[0], s_acc.at[pl.ds(base + kk * B, B)])

        sems = (sem0, sem1)

        def slab0(h, c):
            pltpu.sync_copy(src_hbm.at[sid, pl.ds(h * IDXH, IDXH)], src_idx)
            pltpu.sync_copy(dst_hbm.at[sid, pl.ds(h * IDXH, IDXH)], dst_idx)
            cps = [None, None]
            cps[0] = pltpu.async_copy(
                x_hbm.at[src_idx.at[0]], rows.at[0], sem0)
            for j in range(IDXH):
                b = j % 2
                if j + 1 < IDXH:
                    nb = (j + 1) % 2
                    cps[nb] = pltpu.async_copy(
                        x_hbm.at[src_idx.at[j + 1]], rows.at[nb], sems[nb])
                cps[b].wait()
                pltpu.sync_copy(rows.at[b], s_acc.at[dst_idx.at[j]], add=True)
            return c

        def slab1(h, c):
            pltpu.sync_copy(dst_hbm.at[sid, pl.ds(h * IDXH, IDXH)], dst_idx)
            for j in range(IDXH):
                pltpu.sync_copy(rows.at[0], s_acc.at[dst_idx.at[j]], add=True)
            return c

        @pl.when(cid == 0)
        def _():
            plsc.subcore_barrier()
            lax.fori_loop(0, SPLIT, slab0, 0)
            plsc.subcore_barrier()
            pltpu.sync_copy(s_acc.at[pl.ds(base, ZPT)],
                            out_hbm.at[0, pl.ds(base, ZPT)])

        @pl.when(cid == 1)
        def _():
            ovec = jnp.ones((16,), jnp.float32)

            def orow(i, c):
                for g in range(D // 16):
                    rows[0, i, pl.ds(g * 16, 16)] = ovec
                return c

            lax.fori_loop(0, B, orow, 0)
            plsc.subcore_barrier()
            lax.fori_loop(0, NSLAB, slab1, 0)
            plsc.subcore_barrier()
            pltpu.sync_copy(s_acc.at[pl.ds(base, ZPT)],
                            out_hbm.at[1, pl.ds(base, ZPT)])
            lax.fori_loop(0, B, zrow, 0)
            for kk in range(ZPT // B):
                pltpu.sync_copy(rows.at[0], s_acc.at[pl.ds(base + kk * B, B)])
            plsc.subcore_barrier()
            lax.fori_loop(SPLIT, NSLAB, slab0, 0)
            plsc.subcore_barrier()
            pltpu.sync_copy(s_acc.at[pl.ds(base, ZPT)],
                            out_hbm.at[2, pl.ds(base, ZPT)])

    return k(x, src, dst)


def _tc_body(x_ref, sc_ref, ws_ref, wd_ref, b_ref, o_ref):
    s = sc_ref[0] + sc_ref[2]
    cnt = sc_ref[1]
    mean = s / jnp.maximum(cnt, 1.0)
    o_ref[...] = (
        jnp.dot(mean, ws_ref[...], preferred_element_type=jnp.float32)
        + jnp.dot(x_ref[...], wd_ref[...], preferred_element_type=jnp.float32)
        + b_ref[...]
    )


def _tc_finish(x, sums_counts, ws_t, wd_t, bias):
    R = 1024
    return pl.pallas_call(
        _tc_body,
        grid=(NACC // R,),
        in_specs=[
            pl.BlockSpec((R, D), lambda i: (i, 0)),
            pl.BlockSpec((3, R, D), lambda i: (0, i, 0)),
            pl.BlockSpec((D, D), lambda i: (0, 0)),
            pl.BlockSpec((D, D), lambda i: (0, 0)),
            pl.BlockSpec((1, D), lambda i: (0, 0)),
        ],
        out_specs=pl.BlockSpec((R, D), lambda i: (i, 0)),
        out_shape=jax.ShapeDtypeStruct((N_NODES, D), jnp.float32),
    )(x, sums_counts, ws_t, wd_t, bias)


def kernel(x, edge_index, W_src, b_src, W_dst, b_dst):
    npad = E_PAD - N_EDGES
    src = edge_index[0].astype(jnp.int32)
    dst = edge_index[1].astype(jnp.int32)
    src = jnp.concatenate([src, jnp.zeros((npad,), jnp.int32)])
    dst = jnp.concatenate([dst, jnp.full((npad,), N_NODES, jnp.int32)])
    src = src.reshape(NS, NCHUNK, B)
    dst = dst.reshape(NS, NCHUNK, B)
    sums_counts = _sc_scatter(x, src, dst)
    bias = (b_src + b_dst).reshape(1, D)
    return _tc_finish(x, sums_counts, W_src.T, W_dst.T, bias)


# speedup vs baseline: 3.8950x; 1.0009x over previous
import functools

import jax
import jax.numpy as jnp
from jax import lax
from jax.experimental import pallas as pl
from jax.experimental.pallas import tpu as pltpu
from jax.experimental.pallas import tpu_sc as plsc

N_NODES = 10000
D = 128
N_EDGES = 320000
NC = 2
NS = 16
B = 128
NCHUNK = 160
IDXH = 16
NSLAB = NCHUNK // IDXH
SPLIT = 6
E_PAD = NS * NCHUNK * B
NACC = 10240
ZPT = NACC // NS


def _sc_scatter(x, src, dst):
    mesh = plsc.VectorSubcoreMesh(core_axis_name="c", subcore_axis_name="s")

    @functools.partial(
        pl.kernel,
        out_type=jax.ShapeDtypeStruct((3, NACC, D), jnp.float32),
        mesh=mesh,
        scratch_types=(
            pltpu.VMEM_SHARED((NACC, D), jnp.float32),
            pltpu.VMEM((IDXH, B), jnp.int32),
            pltpu.VMEM((IDXH, B), jnp.int32),
            pltpu.VMEM((2, B, D), jnp.float32),
        ) + (pltpu.SemaphoreType.DMA,) * 8,
    )
    def k(x_hbm, src_hbm, dst_hbm, out_hbm,
          s_acc, src_idx, dst_idx, rows, *sems8):
        cid = lax.axis_index("c")
        sid = lax.axis_index("s")

        zvec = jnp.zeros((16,), jnp.float32)

        def zrow(i, c):
            for g in range(D // 16):
                rows[0, i, pl.ds(g * 16, 16)] = zvec
            return c

        lax.fori_loop(0, B, zrow, 0)
        base = sid * ZPT
        for kk in range(ZPT // B):
            pltpu.sync_copy(rows.at[0], s_acc.at[pl.ds(base + kk * B, B)])

        SUB = 4
        SR = B // SUB
        sems = (sems8[:SUB], sems8[SUB:])

        def fire(j, b):
            return [
                pltpu.async_copy(
                    x_hbm.at[src_idx.at[j, pl.ds(s * SR, SR)]],
                    rows.at[b, pl.ds(s * SR, SR)],
                    sems[b][s])
                for s in range(SUB)
            ]

        def slab0(h, c):
            pltpu.sync_copy(src_hbm.at[sid, pl.ds(h * IDXH, IDXH)], src_idx)
            pltpu.sync_copy(dst_hbm.at[sid, pl.ds(h * IDXH, IDXH)], dst_idx)
            cps = [None, None]
            cps[0] = fire(0, 0)
            for j in range(IDXH):
                b = j % 2
                if j + 1 < IDXH:
                    nb = (j + 1) % 2
                    cps[nb] = fire(j + 1, nb)
                for cp in cps[b]:
                    cp.wait()
                pltpu.sync_copy(rows.at[b], s_acc.at[dst_idx.at[j]], add=True)
            return c

        def slab1(h, c):
            pltpu.sync_copy(dst_hbm.at[sid, pl.ds(h * IDXH, IDXH)], dst_idx)
            for j in range(IDXH):
                pltpu.sync_copy(rows.at[0], s_acc.at[dst_idx.at[j]], add=True)
            return c

        @pl.when(cid == 0)
        def _():
            plsc.subcore_barrier()
            lax.fori_loop(0, SPLIT, slab0, 0)
            plsc.subcore_barrier()
            pltpu.sync_copy(s_acc.at[pl.ds(base, ZPT)],
                            out_hbm.at[0, pl.ds(base, ZPT)])

        @pl.when(cid == 1)
        def _():
            ovec = jnp.ones((16,), jnp.float32)

            def orow(i, c):
                for g in range(D // 16):
                    rows[0, i, pl.ds(g * 16, 16)] = ovec
                return c

            lax.fori_loop(0, B, orow, 0)
            plsc.subcore_barrier()
            lax.fori_loop(0, NSLAB, slab1, 0)
            plsc.subcore_barrier()
            pltpu.sync_copy(s_acc.at[pl.ds(base, ZPT)],
                            out_hbm.at[1, pl.ds(base, ZPT)])
            lax.fori_loop(0, B, zrow, 0)
            for kk in range(ZPT // B):
                pltpu.sync_copy(rows.at[0], s_acc.at[pl.ds(base + kk * B, B)])
            plsc.subcore_barrier()
            lax.fori_loop(SPLIT, NSLAB, slab0, 0)
            plsc.subcore_barrier()
            pltpu.sync_copy(s_acc.at[pl.ds(base, ZPT)],
                            out_hbm.at[2, pl.ds(base, ZPT)])

    return k(x, src, dst)


def _tc_body(x_ref, sc_ref, ws_ref, wd_ref, b_ref, o_ref):
    s = sc_ref[0] + sc_ref[2]
    cnt = sc_ref[1]
    mean = s / jnp.maximum(cnt, 1.0)
    o_ref[...] = (
        jnp.dot(mean, ws_ref[...], preferred_element_type=jnp.float32)
        + jnp.dot(x_ref[...], wd_ref[...], preferred_element_type=jnp.float32)
        + b_ref[...]
    )


def _tc_finish(x, sums_counts, ws_t, wd_t, bias):
    R = 1024
    return pl.pallas_call(
        _tc_body,
        grid=(NACC // R,),
        in_specs=[
            pl.BlockSpec((R, D), lambda i: (i, 0)),
            pl.BlockSpec((3, R, D), lambda i: (0, i, 0)),
            pl.BlockSpec((D, D), lambda i: (0, 0)),
            pl.BlockSpec((D, D), lambda i: (0, 0)),
            pl.BlockSpec((1, D), lambda i: (0, 0)),
        ],
        out_specs=pl.BlockSpec((R, D), lambda i: (i, 0)),
        out_shape=jax.ShapeDtypeStruct((N_NODES, D), jnp.float32),
    )(x, sums_counts, ws_t, wd_t, bias)


def kernel(x, edge_index, W_src, b_src, W_dst, b_dst):
    npad = E_PAD - N_EDGES
    src = edge_index[0].astype(jnp.int32)
    dst = edge_index[1].astype(jnp.int32)
    src = jnp.concatenate([src, jnp.zeros((npad,), jnp.int32)])
    dst = jnp.concatenate([dst, jnp.full((npad,), N_NODES, jnp.int32)])
    src = src.reshape(NS, NCHUNK, B)
    dst = dst.reshape(NS, NCHUNK, B)
    sums_counts = _sc_scatter(x, src, dst)
    bias = (b_src + b_dst).reshape(1, D)
    return _tc_finish(x, sums_counts, W_src.T, W_dst.T, bias)
